# baseline (speedup 1.0000x reference)
import functools

import jax
import jax.numpy as jnp
import numpy as np
from jax import lax
from jax.experimental import pallas as pl
from jax.experimental.pallas import tpu as pltpu
from jax.experimental.pallas import tpu_sc as plsc

OUT = 7
SR = 2
G = OUT * SR
CELLS = OUT * OUT
TERMS = SR * SR * 4
NPTS = CELLS * TERMS
C = 96
NB = 2048
NREAL = 2000
BLK = 256
NW = 32
BPW = NB // NW
N0PT = 104
N1PT = 24
N0TOT = 16 * N0PT
CHUNK = 112
NCHUNK = NPTS // CHUNK

_HS = np.array([128.0, 64.0, 32.0, 16.0], np.float32)
_SCALES = np.array([0.25, 0.125, 0.0625, 0.03125], np.float32)
_BASE = np.array([0, 2 * 128 * 128, 2 * 128 * 128 + 2 * 64 * 64,
                  2 * 128 * 128 + 2 * 64 * 64 + 2 * 32 * 32], np.int32)
_HH = np.array([128 * 128, 64 * 64, 32 * 32, 16 * 16], np.int32)

_OFF = (np.arange(OUT, dtype=np.float32)[:, None]
        + (np.arange(SR, dtype=np.float32)[None, :] + 0.5) / SR).reshape(-1)
_r = np.arange(NPTS)
_corner = _r % 4
_sub = (_r // 4) % 4
_cell = _r // 16
_p = _cell // OUT
_q = _cell % OUT
_gy = 2 * _p + _sub // 2
_gx = 2 * _q + _sub % 2
GYO = _OFF[_gy].reshape(1, NPTS)
GXO = _OFF[_gx].reshape(1, NPTS)
CYM = (_corner // 2 == 1).reshape(1, NPTS)
CXM = (_corner % 2 == 1).reshape(1, NPTS)


def _coord_body(boxes_ref, gyo_ref, gxo_ref, cym_ref, cxm_ref, idx_ref, wts_ref):
    pid = pl.program_id(0)
    x1 = boxes_ref[:, 0:1]
    y1 = boxes_ref[:, 1:2]
    x2 = boxes_ref[:, 2:3]
    y2 = boxes_ref[:, 3:4]
    gidx = pid * BLK + lax.broadcasted_iota(jnp.int32, (BLK, 1), 0)
    bid = (gidx >= 1000).astype(jnp.int32)

    area = (x2 - x1) * (y2 - y1)
    s = jnp.sqrt(area)
    tgt = jnp.floor(4.0 + jnp.log2(s / 224.0) + 1e-6)
    lvl = jnp.clip(tgt, 2.0, 5.0).astype(jnp.int32) - 2
    scale = jnp.where(lvl == 0, 0.25,
                      jnp.where(lvl == 1, 0.125,
                                jnp.where(lvl == 2, 0.0625, 0.03125)))
    hl = jnp.where(lvl == 0, 128.0,
                   jnp.where(lvl == 1, 64.0,
                             jnp.where(lvl == 2, 32.0, 16.0)))
    hli = hl.astype(jnp.int32)
    base = jnp.where(lvl == 0, _BASE[0],
                     jnp.where(lvl == 1, _BASE[1],
                               jnp.where(lvl == 2, _BASE[2], _BASE[3])))
    hh = jnp.where(lvl == 0, _HH[0],
                   jnp.where(lvl == 1, _HH[1],
                             jnp.where(lvl == 2, _HH[2], _HH[3])))

    x1s = x1 * scale
    y1s = y1 * scale
    roi_w = jnp.maximum(x2 * scale - x1s, 1.0)
    roi_h = jnp.maximum(y2 * scale - y1s, 1.0)
    bw = roi_w / OUT
    bh = roi_h / OUT

    Y = y1s + bh * gyo_ref[...]
    X = x1s + bw * gxo_ref[...]
    valid = ((Y >= -1.0) & (Y <= hl) & (X >= -1.0) & (X <= hl))
    Yc = jnp.clip(Y, 0.0, hl - 1.0)
    Xc = jnp.clip(X, 0.0, hl - 1.0)
    y0f = jnp.floor(Yc)
    x0f = jnp.floor(Xc)
    y0i = y0f.astype(jnp.int32)
    x0i = x0f.astype(jnp.int32)
    y1i = jnp.minimum(y0i + 1, hli - 1)
    x1i = jnp.minimum(x0i + 1, hli - 1)
    ly = Yc - y0f
    lx = Xc - x0f

    cym = cym_ref[...] != 0
    cxm = cxm_ref[...] != 0
    yi = jnp.where(cym, y1i, y0i)
    xi = jnp.where(cxm, x1i, x0i)
    wy = jnp.where(cym, ly, 1.0 - ly)
    wx = jnp.where(cxm, lx, 1.0 - lx)

    idx_ref[...] = base + bid * hh + yi * hli + xi
    wts_ref[...] = wy * wx * valid.astype(jnp.float32) * 0.25


def _coords(boxes, interpret=False):
    tbl_spec = pl.BlockSpec((1, NPTS), lambda i: (0, 0))
    return pl.pallas_call(
        _coord_body,
        grid=(NB // BLK,),
        in_specs=[pl.BlockSpec((BLK, 4), lambda i: (i, 0)),
                  tbl_spec, tbl_spec, tbl_spec, tbl_spec],
        out_specs=[pl.BlockSpec((BLK, NPTS), lambda i: (i, 0)),
                   pl.BlockSpec((BLK, NPTS), lambda i: (i, 0))],
        out_shape=[jax.ShapeDtypeStruct((NB, NPTS), jnp.int32),
                   jax.ShapeDtypeStruct((NB, NPTS), jnp.float32)],
        interpret=interpret,
    )(boxes, jnp.asarray(GYO), jnp.asarray(GXO),
      jnp.asarray(CYM, jnp.int32), jnp.asarray(CXM, jnp.int32))


def _sc_pool(table, idx, wts):
    mesh = plsc.VectorSubcoreMesh(core_axis_name="c", subcore_axis_name="s")

    @functools.partial(
        pl.kernel,
        mesh=mesh,
        compiler_params=pltpu.CompilerParams(use_tc_tiling_on_sc=False),
        out_type=jax.ShapeDtypeStruct((NB * CELLS * C,), jnp.float32),
        scratch_types=[
            pltpu.VMEM((2, NCHUNK, CHUNK), jnp.int32),
            pltpu.VMEM((2, CELLS, TERMS), jnp.float32),
            pltpu.VMEM((NPTS, C), jnp.float32),
            pltpu.VMEM((2, CELLS * C), jnp.float32),
        ] + [pltpu.SemaphoreType.DMA] * (NCHUNK + 4),
    )
    def sck(table_hbm, idx_hbm, wts_hbm, out_hbm, idx_v, wts_v, rows_v, out_v,
            sem0, sem1, sem2, sem3, sem4, sem5, sem6, sem_pfi, sem_pfw,
            sem_o0, sem_o1):
        semk = [sem0, sem1, sem2, sem3, sem4, sem5, sem6]
        sem_o = [sem_o0, sem_o1]
        sid = lax.axis_index("s")
        cid = lax.axis_index("c")
        nb_t = jnp.where(cid == 0, N0PT, N1PT)
        base = jnp.where(cid == 0, sid * N0PT, N0TOT + sid * N1PT)

        def chunk_src(s, k):
            return table_hbm.at[idx_v.at[s].at[k]]

        def chunk_dst(k):
            return rows_v.at[pl.ds(k * CHUNK, CHUNK)]

        pltpu.sync_copy(idx_hbm.at[base], idx_v.at[0])
        pltpu.sync_copy(wts_hbm.at[base], wts_v.at[0])
        for k in range(NCHUNK):
            pltpu.async_copy(chunk_src(0, k), chunk_dst(k), semk[k])

        def half(i, s):
            gb = base + i

            @pl.when(i + 1 < nb_t)
            def _():
                pltpu.async_copy(idx_hbm.at[gb + 1], idx_v.at[1 - s], sem_pfi)
                pltpu.async_copy(wts_hbm.at[gb + 1], wts_v.at[1 - s], sem_pfw)

            @pl.when(i >= 2)
            def _():
                pltpu.make_async_copy(
                    out_v.at[s], out_hbm.at[pl.ds((gb - 2) * CELLS * C,
                                                  CELLS * C)],
                    sem_o[s]).wait()

            for k in range(NCHUNK):
                pltpu.make_async_copy(chunk_src(s, k), chunk_dst(k),
                                      semk[k]).wait()

                @pl.loop(0, OUT)
                def _cells(j):
                    c = k * OUT + j
                    wvec = wts_v[s, c, pl.ds(0, TERMS)]
                    accs = None
                    for t in range(TERMS):
                        w = wvec[t]
                        cur = [w * rows_v[c * TERMS + t, pl.ds(v * 16, 16)]
                               for v in range(C // 16)]
                        if accs is None:
                            accs = cur
                        else:
                            accs = [a + x for a, x in zip(accs, cur)]
                    for v in range(C // 16):
                        out_v[s, pl.ds(c * C + v * 16, 16)] = accs[v]

                if k == 0:
                    @pl.when(i + 1 < nb_t)
                    def _():
                        pltpu.make_async_copy(idx_hbm.at[gb + 1],
                                              idx_v.at[1 - s], sem_pfi).wait()
                        pltpu.make_async_copy(wts_hbm.at[gb + 1],
                                              wts_v.at[1 - s], sem_pfw).wait()

                @pl.when(i + 1 < nb_t)
                def _():
                    pltpu.async_copy(chunk_src(1 - s, k), chunk_dst(k),
                                     semk[k])

            pltpu.async_copy(out_v.at[s],
                             out_hbm.at[pl.ds(gb * CELLS * C, CELLS * C)],
                             sem_o[s])

        @pl.loop(0, nb_t, step=2)
        def _box(i):
            half(i, 0)
            half(i + 1, 1)

        pltpu.make_async_copy(
            out_v.at[0],
            out_hbm.at[pl.ds((base + nb_t - 2) * CELLS * C, CELLS * C)],
            sem_o[0]).wait()
        pltpu.make_async_copy(
            out_v.at[1],
            out_hbm.at[pl.ds((base + nb_t - 1) * CELLS * C, CELLS * C)],
            sem_o[1]).wait()

    return sck(table, idx, wts)


def kernel(feat0, feat1, feat2, feat3, boxes0, boxes1):
    table = jnp.concatenate(
        [jnp.transpose(f, (0, 2, 3, 1)).reshape(-1, C)
         for f in (feat0, feat1, feat2, feat3)], axis=0)
    boxes = jnp.concatenate(
        [boxes0, boxes1,
         jnp.zeros((NB - NREAL, 4), jnp.float32)], axis=0)
    idx, wts = _coords(boxes)
    pooled = _sc_pool(table,
                      idx.reshape(NB, NCHUNK, CHUNK),
                      wts.reshape(NB, CELLS, TERMS))
    pooled = pooled.reshape(NB, CELLS, C)
    out = pooled[:NREAL].reshape(NREAL, OUT, OUT, C)
    return jnp.transpose(out, (0, 3, 1, 2))


# speedup vs baseline: 1.2428x; 1.2428x over previous
import functools

import jax
import jax.numpy as jnp
import numpy as np
from jax import lax
from jax.experimental import pallas as pl
from jax.experimental.pallas import tpu as pltpu
from jax.experimental.pallas import tpu_sc as plsc

OUT = 7
SR = 2
G = OUT * SR
CELLS = OUT * OUT
TERMS = SR * SR * 4
NPTS = CELLS * TERMS
C = 96
NB = 2048
NREAL = 2000
BLK = 256
NW = 32
BPW = NB // NW
N0PT = 104
N1PT = 24
N0TOT = 16 * N0PT
CHUNK = 112
NCHUNK = NPTS // CHUNK

_HS = np.array([128.0, 64.0, 32.0, 16.0], np.float32)
_SCALES = np.array([0.25, 0.125, 0.0625, 0.03125], np.float32)
_BASE = np.array([0, 2 * 128 * 128, 2 * 128 * 128 + 2 * 64 * 64,
                  2 * 128 * 128 + 2 * 64 * 64 + 2 * 32 * 32], np.int32)
_HH = np.array([128 * 128, 64 * 64, 32 * 32, 16 * 16], np.int32)

_OFF = (np.arange(OUT, dtype=np.float32)[:, None]
        + (np.arange(SR, dtype=np.float32)[None, :] + 0.5) / SR).reshape(-1)
_r = np.arange(NPTS)
_corner = _r % 4
_sub = (_r // 4) % 4
_cell = _r // 16
_p = _cell // OUT
_q = _cell % OUT
_gy = 2 * _p + _sub // 2
_gx = 2 * _q + _sub % 2
GYO = _OFF[_gy].reshape(1, NPTS)
GXO = _OFF[_gx].reshape(1, NPTS)
CYM = (_corner // 2 == 1).reshape(1, NPTS)
CXM = (_corner % 2 == 1).reshape(1, NPTS)

_PERM = np.concatenate(
    [np.stack([np.arange(16), np.arange(16) + 16], axis=1).reshape(-1) + 32 * u
     for u in range(3)])


def _coord_body(boxes_ref, gyo_ref, gxo_ref, cym_ref, cxm_ref, idx_ref, wts_ref):
    pid = pl.program_id(0)
    x1 = boxes_ref[:, 0:1]
    y1 = boxes_ref[:, 1:2]
    x2 = boxes_ref[:, 2:3]
    y2 = boxes_ref[:, 3:4]
    gidx = pid * BLK + lax.broadcasted_iota(jnp.int32, (BLK, 1), 0)
    bid = (gidx >= 1000).astype(jnp.int32)

    area = (x2 - x1) * (y2 - y1)
    s = jnp.sqrt(area)
    tgt = jnp.floor(4.0 + jnp.log2(s / 224.0) + 1e-6)
    lvl = jnp.clip(tgt, 2.0, 5.0).astype(jnp.int32) - 2
    scale = jnp.where(lvl == 0, 0.25,
                      jnp.where(lvl == 1, 0.125,
                                jnp.where(lvl == 2, 0.0625, 0.03125)))
    hl = jnp.where(lvl == 0, 128.0,
                   jnp.where(lvl == 1, 64.0,
                             jnp.where(lvl == 2, 32.0, 16.0)))
    hli = hl.astype(jnp.int32)
    base = jnp.where(lvl == 0, _BASE[0],
                     jnp.where(lvl == 1, _BASE[1],
                               jnp.where(lvl == 2, _BASE[2], _BASE[3])))
    hh = jnp.where(lvl == 0, _HH[0],
                   jnp.where(lvl == 1, _HH[1],
                             jnp.where(lvl == 2, _HH[2], _HH[3])))

    x1s = x1 * scale
    y1s = y1 * scale
    roi_w = jnp.maximum(x2 * scale - x1s, 1.0)
    roi_h = jnp.maximum(y2 * scale - y1s, 1.0)
    bw = roi_w / OUT
    bh = roi_h / OUT

    Y = y1s + bh * gyo_ref[...]
    X = x1s + bw * gxo_ref[...]
    valid = ((Y >= -1.0) & (Y <= hl) & (X >= -1.0) & (X <= hl))
    Yc = jnp.clip(Y, 0.0, hl - 1.0)
    Xc = jnp.clip(X, 0.0, hl - 1.0)
    y0f = jnp.floor(Yc)
    x0f = jnp.floor(Xc)
    y0i = y0f.astype(jnp.int32)
    x0i = x0f.astype(jnp.int32)
    y1i = jnp.minimum(y0i + 1, hli - 1)
    x1i = jnp.minimum(x0i + 1, hli - 1)
    ly = Yc - y0f
    lx = Xc - x0f

    cym = cym_ref[...] != 0
    cxm = cxm_ref[...] != 0
    yi = jnp.where(cym, y1i, y0i)
    xi = jnp.where(cxm, x1i, x0i)
    wy = jnp.where(cym, ly, 1.0 - ly)
    wx = jnp.where(cxm, lx, 1.0 - lx)

    idx_ref[...] = base + bid * hh + yi * hli + xi
    wts_ref[...] = wy * wx * valid.astype(jnp.float32) * 0.25


def _coords(boxes, interpret=False):
    tbl_spec = pl.BlockSpec((1, NPTS), lambda i: (0, 0))
    return pl.pallas_call(
        _coord_body,
        grid=(NB // BLK,),
        in_specs=[pl.BlockSpec((BLK, 4), lambda i: (i, 0)),
                  tbl_spec, tbl_spec, tbl_spec, tbl_spec],
        out_specs=[pl.BlockSpec((BLK, NPTS), lambda i: (i, 0)),
                   pl.BlockSpec((BLK, NPTS), lambda i: (i, 0))],
        out_shape=[jax.ShapeDtypeStruct((NB, NPTS), jnp.int32),
                   jax.ShapeDtypeStruct((NB, NPTS), jnp.float32)],
        interpret=interpret,
    )(boxes, jnp.asarray(GYO), jnp.asarray(GXO),
      jnp.asarray(CYM, jnp.int32), jnp.asarray(CXM, jnp.int32))


def _sc_pool(table, idx, wts):
    mesh = plsc.VectorSubcoreMesh(core_axis_name="c", subcore_axis_name="s")

    @functools.partial(
        pl.kernel,
        mesh=mesh,
        compiler_params=pltpu.CompilerParams(use_tc_tiling_on_sc=False,
                                             needs_layout_passes=False),
        out_type=jax.ShapeDtypeStruct((NB * CELLS * C,), jnp.float32),
        scratch_types=[
            pltpu.VMEM((2, NCHUNK, CHUNK), jnp.int32),
            pltpu.VMEM((2, CELLS, TERMS), jnp.float32),
            pltpu.VMEM((NPTS, C), jnp.bfloat16),
            pltpu.VMEM((2, CELLS * C), jnp.float32),
        ] + [pltpu.SemaphoreType.DMA] * (NCHUNK + 4),
    )
    def sck(table_hbm, idx_hbm, wts_hbm, out_hbm, idx_v, wts_v, rows_v, out_v,
            sem0, sem1, sem2, sem3, sem4, sem5, sem6, sem_pfi, sem_pfw,
            sem_o0, sem_o1):
        semk = [sem0, sem1, sem2, sem3, sem4, sem5, sem6]
        sem_o = [sem_o0, sem_o1]
        sid = lax.axis_index("s")
        cid = lax.axis_index("c")
        nb_t = jnp.where(cid == 0, N0PT, N1PT)
        base = jnp.where(cid == 0, sid * N0PT, N0TOT + sid * N1PT)

        def chunk_src(s, k):
            return table_hbm.at[idx_v.at[s].at[k]]

        def chunk_dst(k):
            return rows_v.at[pl.ds(k * CHUNK, CHUNK)]

        pltpu.sync_copy(idx_hbm.at[base], idx_v.at[0])
        pltpu.sync_copy(wts_hbm.at[base], wts_v.at[0])
        for k in range(NCHUNK):
            pltpu.async_copy(chunk_src(0, k), chunk_dst(k), semk[k])

        def half(i, s):
            gb = base + i

            @pl.when(i + 1 < nb_t)
            def _():
                pltpu.async_copy(idx_hbm.at[gb + 1], idx_v.at[1 - s], sem_pfi)
                pltpu.async_copy(wts_hbm.at[gb + 1], wts_v.at[1 - s], sem_pfw)

            @pl.when(i >= 2)
            def _():
                pltpu.make_async_copy(
                    out_v.at[s], out_hbm.at[pl.ds((gb - 2) * CELLS * C,
                                                  CELLS * C)],
                    sem_o[s]).wait()

            for k in range(NCHUNK):
                pltpu.make_async_copy(chunk_src(s, k), chunk_dst(k),
                                      semk[k]).wait()

                @pl.loop(0, OUT)
                def _cells(j):
                    c = k * OUT + j
                    wvec = wts_v[s, c, pl.ds(0, TERMS)]
                    accs = None
                    for t in range(TERMS):
                        w = wvec[t]
                        cur = []
                        for u in range(C // 32):
                            packed = rows_v[c * TERMS + t, pl.ds(u * 32, 32)]
                            a, b = plsc.unpack(
                                packed, format=plsc.PackFormat.INTERLEAVED,
                                preferred_element_type=jnp.float32)
                            cur.append(w * a)
                            cur.append(w * b)
                        if accs is None:
                            accs = cur
                        else:
                            accs = [p + q for p, q in zip(accs, cur)]
                    for v in range(C // 16):
                        out_v[s, pl.ds(c * C + v * 16, 16)] = accs[v]

                if k == 0:
                    @pl.when(i + 1 < nb_t)
                    def _():
                        pltpu.make_async_copy(idx_hbm.at[gb + 1],
                                              idx_v.at[1 - s], sem_pfi).wait()
                        pltpu.make_async_copy(wts_hbm.at[gb + 1],
                                              wts_v.at[1 - s], sem_pfw).wait()

                @pl.when(i + 1 < nb_t)
                def _():
                    pltpu.async_copy(chunk_src(1 - s, k), chunk_dst(k),
                                     semk[k])

            pltpu.async_copy(out_v.at[s],
                             out_hbm.at[pl.ds(gb * CELLS * C, CELLS * C)],
                             sem_o[s])

        @pl.loop(0, nb_t, step=2)
        def _box(i):
            half(i, 0)
            half(i + 1, 1)

        pltpu.make_async_copy(
            out_v.at[0],
            out_hbm.at[pl.ds((base + nb_t - 2) * CELLS * C, CELLS * C)],
            sem_o[0]).wait()
        pltpu.make_async_copy(
            out_v.at[1],
            out_hbm.at[pl.ds((base + nb_t - 1) * CELLS * C, CELLS * C)],
            sem_o[1]).wait()

    return sck(table, idx, wts)


def kernel(feat0, feat1, feat2, feat3, boxes0, boxes1):
    perm = jnp.asarray(_PERM)
    table = jnp.concatenate(
        [jnp.transpose(f, (0, 2, 3, 1)).reshape(-1, C)
         for f in (feat0, feat1, feat2, feat3)], axis=0)
    table = table[:, perm].astype(jnp.bfloat16)
    boxes = jnp.concatenate(
        [boxes0, boxes1,
         jnp.zeros((NB - NREAL, 4), jnp.float32)], axis=0)
    idx, wts = _coords(boxes)
    pooled = _sc_pool(table,
                      idx.reshape(NB, NCHUNK, CHUNK),
                      wts.reshape(NB, CELLS, TERMS))
    pooled = pooled.reshape(NB, CELLS, C)
    out = pooled[:NREAL].reshape(NREAL, OUT, OUT, C)
    return jnp.transpose(out, (0, 3, 1, 2))
